# Initial kernel scaffold; baseline (speedup 1.0000x reference)
import functools

import jax
import jax.numpy as jnp
import numpy as np
from jax import lax
from jax.experimental import pallas as pl
from jax.experimental.pallas import tpu as pltpu
from jax.experimental.pallas import tpu_sc as plsc

VOCAB = 8192
D_MODEL = 768
BATCH = 4

NUM_CORES = 2
NUM_SUBCORES = 16
NUM_WORKERS = NUM_CORES * NUM_SUBCORES
S_PER_WORKER = VOCAB // NUM_WORKERS
CHUNK = 64
CHUNKS_PER_WORKER = S_PER_WORKER // CHUNK
LANES = 16
VREGS_PER_ROW = D_MODEL // LANES


def _positional_encodings() -> np.ndarray:
    pos = np.arange(VOCAB, dtype=np.float32)[:, None]
    i = np.arange(D_MODEL, dtype=np.float32)[None, :]
    angle = (pos / np.power(np.float32(10000.0), 2.0 * i / np.float32(D_MODEL),
                            dtype=np.float32)).astype(np.float32)
    even = (np.arange(D_MODEL)[None, :] % 2) == 0
    return np.where(even, np.sin(angle), np.cos(angle)).astype(np.float32)


_PE = jnp.asarray(_positional_encodings())


def _sc_body(idx_hbm, pe_hbm, table_hbm, out_hbm, idx_v, pe_v, rows_v, sem):
    wid = lax.axis_index("s") * NUM_CORES + lax.axis_index("c")
    s_base = wid * S_PER_WORKER
    pltpu.sync_copy(idx_hbm.at[wid], idx_v)

    def chunk_body(j, carry):
        s0 = s_base + j * CHUNK
        pltpu.sync_copy(pe_hbm.at[pl.ds(s0, CHUNK)], pe_v)

        def batch_body(b, carry2):
            pltpu.async_copy(table_hbm.at[idx_v.at[j, b]], rows_v, sem).wait()

            def row_body(c, carry3):
                for k in range(VREGS_PER_ROW):
                    sl = pl.ds(k * LANES, LANES)
                    plsc.addupdate(rows_v.at[c, sl], pe_v[c, sl])
                return carry3

            lax.fori_loop(0, CHUNK, row_body, 0, unroll=False)
            out_row = b * VOCAB + s0
            pltpu.sync_copy(rows_v, out_hbm.at[pl.ds(out_row, CHUNK)])
            return carry2

        lax.fori_loop(0, BATCH, batch_body, 0, unroll=False)
        return carry

    lax.fori_loop(0, CHUNKS_PER_WORKER, chunk_body, 0, unroll=False)


@functools.partial(
    pl.kernel,
    out_type=jax.ShapeDtypeStruct((BATCH * VOCAB, D_MODEL), jnp.float32),
    mesh=plsc.VectorSubcoreMesh(
        core_axis_name="c", subcore_axis_name="s",
        num_cores=NUM_CORES, num_subcores=NUM_SUBCORES),
    scratch_types=[
        pltpu.VMEM((CHUNKS_PER_WORKER, BATCH, CHUNK), jnp.int32),
        pltpu.VMEM((CHUNK, D_MODEL), jnp.float32),
        pltpu.VMEM((CHUNK, D_MODEL), jnp.float32),
        pltpu.SemaphoreType.DMA,
    ],
)
def _sc_embed(idx_hbm, pe_hbm, table_hbm, out_hbm, idx_v, pe_v, rows_v, sem):
    _sc_body(idx_hbm, pe_hbm, table_hbm, out_hbm, idx_v, pe_v, rows_v, sem)


def kernel(x, table):
    idx = x.reshape(BATCH, NUM_WORKERS, CHUNKS_PER_WORKER, CHUNK)
    idx = idx.transpose(1, 2, 0, 3)
    out = _sc_embed(idx, _PE, table)
    return out.reshape(BATCH, VOCAB, D_MODEL)


# speedup vs baseline: 1.8076x; 1.8076x over previous
import functools

import jax
import jax.numpy as jnp
import numpy as np
from jax import lax
from jax.experimental import pallas as pl
from jax.experimental.pallas import tpu as pltpu
from jax.experimental.pallas import tpu_sc as plsc

VOCAB = 8192
D_MODEL = 768
BATCH = 4

NUM_CORES = 2
NUM_SUBCORES = 16
NUM_WORKERS = NUM_CORES * NUM_SUBCORES
S_PER_WORKER = VOCAB // NUM_WORKERS
CHUNK = 64
CHUNKS_PER_WORKER = S_PER_WORKER // CHUNK
LANES = 16
VREGS_PER_ROW = D_MODEL // LANES


def _positional_encodings() -> np.ndarray:
    pos = np.arange(VOCAB, dtype=np.float32)[:, None]
    i = np.arange(D_MODEL, dtype=np.float32)[None, :]
    angle = (pos / np.power(np.float32(10000.0), 2.0 * i / np.float32(D_MODEL),
                            dtype=np.float32)).astype(np.float32)
    even = (np.arange(D_MODEL)[None, :] % 2) == 0
    return np.where(even, np.sin(angle), np.cos(angle)).astype(np.float32)


_PE = _positional_encodings()


def _sc_body(idx_hbm, pe_hbm, table_hbm, out_hbm, idx_v, pe_v, rows_v, sem):
    wid = lax.axis_index("s") * NUM_CORES + lax.axis_index("c")
    s_base = wid * S_PER_WORKER
    pltpu.sync_copy(idx_hbm.at[wid], idx_v)

    def chunk_body(j, carry):
        s0 = s_base + j * CHUNK
        pltpu.sync_copy(pe_hbm.at[pl.ds(s0, CHUNK)], pe_v)

        def batch_body(b, carry2):
            pltpu.async_copy(table_hbm.at[idx_v.at[j, b]], rows_v, sem).wait()

            def row_body(c, carry3):
                for k in range(VREGS_PER_ROW):
                    sl = pl.ds(k * LANES, LANES)
                    plsc.addupdate(rows_v.at[c, sl], pe_v[c, sl])
                return carry3

            lax.fori_loop(0, CHUNK, row_body, 0, unroll=False)
            out_row = b * VOCAB + s0
            pltpu.sync_copy(rows_v, out_hbm.at[pl.ds(out_row, CHUNK)])
            return carry2

        lax.fori_loop(0, BATCH, batch_body, 0, unroll=False)
        return carry

    lax.fori_loop(0, CHUNKS_PER_WORKER, chunk_body, 0, unroll=False)


@functools.partial(
    pl.kernel,
    out_type=jax.ShapeDtypeStruct((BATCH * VOCAB, D_MODEL), jnp.float32),
    mesh=plsc.VectorSubcoreMesh(
        core_axis_name="c", subcore_axis_name="s",
        num_cores=NUM_CORES, num_subcores=NUM_SUBCORES),
    scratch_types=[
        pltpu.VMEM((CHUNKS_PER_WORKER, BATCH, CHUNK), jnp.int32),
        pltpu.VMEM((CHUNK, D_MODEL), jnp.float32),
        pltpu.VMEM((CHUNK, D_MODEL), jnp.float32),
        pltpu.SemaphoreType.DMA,
    ],
)
def _sc_embed(idx_hbm, pe_hbm, table_hbm, out_hbm, idx_v, pe_v, rows_v, sem):
    _sc_body(idx_hbm, pe_hbm, table_hbm, out_hbm, idx_v, pe_v, rows_v, sem)


def kernel(x, table):
    idx = x.reshape(BATCH, NUM_WORKERS, CHUNKS_PER_WORKER, CHUNK)
    idx = idx.transpose(1, 2, 0, 3)
    out = _sc_embed(idx, _PE, table)
    return out.reshape(BATCH, VOCAB, D_MODEL)


# speedup vs baseline: 2.1998x; 1.2170x over previous
import functools

import jax
import jax.numpy as jnp
import numpy as np
from jax import lax
from jax.experimental import pallas as pl
from jax.experimental.pallas import tpu as pltpu
from jax.experimental.pallas import tpu_sc as plsc

VOCAB = 8192
D_MODEL = 768
BATCH = 4

NUM_CORES = 2
NUM_SUBCORES = 16
NUM_WORKERS = NUM_CORES * NUM_SUBCORES
S_PER_WORKER = VOCAB // NUM_WORKERS
CHUNK = 16
N_CHUNKS = S_PER_WORKER // CHUNK
NBUF = 4
LANES = 16
VREGS_PER_ROW = D_MODEL // LANES


def _positional_encodings() -> np.ndarray:
    pos = np.arange(VOCAB, dtype=np.float32)[:, None]
    i = np.arange(D_MODEL, dtype=np.float32)[None, :]
    angle = (pos / np.power(np.float32(10000.0), 2.0 * i / np.float32(D_MODEL),
                            dtype=np.float32)).astype(np.float32)
    even = (np.arange(D_MODEL)[None, :] % 2) == 0
    return np.where(even, np.sin(angle), np.cos(angle)).astype(np.float32)


_PE = _positional_encodings()


def _sc_body(idx_hbm, pe_hbm, table_hbm, out_hbm, idx_v, pe_v, rows_v,
             g0, g1, g2, g3, t0, t1, t2, t3, p0, p1):
    g_sems = (g0, g1, g2, g3)
    st_sems = (t0, t1, t2, t3)
    pe_sems = (p0, p1)

    wid = lax.axis_index("s") * NUM_CORES + lax.axis_index("c")
    s_base = wid * S_PER_WORKER
    pltpu.sync_copy(idx_hbm.at[wid], idx_v)

    def pe_copy(chunk, par):
        src = pe_hbm.at[pl.ds(s_base + chunk * CHUNK, CHUNK)]
        return pltpu.make_async_copy(src, pe_v.at[par], pe_sems[par])

    def gather_copy(chunk, b, buf):
        src = table_hbm.at[idx_v.at[chunk, b]]
        return pltpu.make_async_copy(src, rows_v.at[buf], g_sems[buf])

    def store_copy(chunk, b, buf):
        dst = out_hbm.at[pl.ds(b * VOCAB + s_base + chunk * CHUNK, CHUNK)]
        return pltpu.make_async_copy(rows_v.at[buf], dst, st_sems[buf])

    pe_copy(0, 0).start()
    gather_copy(0, 0, 0).start()
    gather_copy(0, 1, 1).start()

    def loop_body(r, carry):
        for w in range(8):
            j = 2 * r + (w // 4)
            b = w % 4
            buf = w % 4
            par = w // 4

            if w == 0:
                pe_copy(2 * r + 1, 1).start()
                pe_copy(0, 0).wait()
            if w == 4:
                @pl.when(r < N_CHUNKS // 2 - 1)
                def _():
                    pe_copy(2 * r + 2, 0).start()
                pe_copy(0, 1).wait()

            gather_copy(0, 0, buf).wait()

            def row_body(c, carry2):
                for k in range(VREGS_PER_ROW):
                    sl = pl.ds(k * LANES, LANES)
                    plsc.addupdate(rows_v.at[buf, c, sl], pe_v[par, c, sl])
                return carry2

            lax.fori_loop(0, CHUNK, row_body, 0, unroll=False)

            store_copy(j, b, buf).start()

            nbuf = (w + 2) % NBUF
            b2 = (w + 2) % 4
            if w < 2:
                @pl.when(r > 0)
                def _():
                    store_copy(0, 0, nbuf).wait()
            else:
                store_copy(0, 0, nbuf).wait()
            if w < 6:
                gather_copy(2 * r + (w + 2) // 4, b2, nbuf).start()
            else:
                @pl.when(r < N_CHUNKS // 2 - 1)
                def _():
                    gather_copy(2 * r + 2, b2, nbuf).start()
        return carry

    lax.fori_loop(0, N_CHUNKS // 2, loop_body, 0, unroll=False)

    store_copy(0, 0, 2).wait()
    store_copy(0, 0, 3).wait()


@functools.partial(
    pl.kernel,
    out_type=jax.ShapeDtypeStruct((BATCH * VOCAB, D_MODEL), jnp.float32),
    mesh=plsc.VectorSubcoreMesh(
        core_axis_name="c", subcore_axis_name="s",
        num_cores=NUM_CORES, num_subcores=NUM_SUBCORES),
    scratch_types=[
        pltpu.VMEM((N_CHUNKS, BATCH, CHUNK), jnp.int32),
        pltpu.VMEM((2, CHUNK, D_MODEL), jnp.float32),
        pltpu.VMEM((NBUF, CHUNK, D_MODEL), jnp.float32),
    ] + [pltpu.SemaphoreType.DMA] * 10,
)
def _sc_embed(*refs):
    _sc_body(*refs)


def kernel(x, table):
    idx = x.reshape(BATCH, NUM_WORKERS, N_CHUNKS, CHUNK)
    idx = idx.transpose(1, 2, 0, 3)
    out = _sc_embed(idx, _PE, table)
    return out.reshape(BATCH, VOCAB, D_MODEL)
